# Initial kernel scaffold; baseline (speedup 1.0000x reference)
import jax
import jax.numpy as jnp
from jax.experimental import pallas as pl


def kernel(features, src, dst, e_params, alpha, attention, labels_init, labels_one_hot, byte_idx_train, W1, b1, W2, b2):
    raise NotImplementedError("write your pallas kernel here")


# speedup vs baseline: 23.5504x; 23.5504x over previous
import functools

import jax
import jax.numpy as jnp
from jax import lax
from jax.experimental import pallas as pl
from jax.experimental.pallas import tpu as pltpu
from jax.experimental.pallas import tpu_sc as plsc

N_NODES = 50000
N_PAD = 50048
E_EDGES = 800000
C_DIM = 32
D_DIM = 128
H_DIM = 64
NG = 2
NL = 2

E_PAD = 819200
KB = 128
KC = 512
KR = KC // KB
EPT = E_PAD // 16
NCHUNK = EPT // KC
ROWS_PT = E_PAD // KB // 16

NB_A = 3200
NB_B = 3072
SZS_A = (512, 512, 512, 512, 512, 512, 128)
SZS_B = (512, 512, 512, 512, 512, 512)

_mesh = plsc.VectorSubcoreMesh(core_axis_name="c", subcore_axis_name="s")


def _zero_rows(buf, nrows):
    z = jnp.zeros((16,), jnp.float32)

    def body(i, _):
        buf[i, pl.ds(0, 16)] = z
        buf[i, pl.ds(16, 16)] = z
        return 0

    lax.fori_loop(0, nrows, body, 0)


def _per_tile(t, plan):
    @pl.when(t < 7)
    def _():
        plan(t * NB_A, SZS_A)

    @pl.when(t >= 7)
    def _():
        plan(7 * NB_A + (t - 7) * NB_B, SZS_B)


def _den_kernel(dst3, e_all, dd_hbm, den_sh, dst_v, e0_v, e1_v, rows_v):
    g = lax.axis_index("c")
    t = lax.axis_index("s")
    base_rows = t * ROWS_PT

    _zero_rows(rows_v, KC)

    def _zero_plan(nb, sizes):
        off = 0
        for sz in sizes:
            pltpu.sync_copy(rows_v.at[pl.ds(0, sz)],
                            den_sh.at[pl.ds(nb + off, sz)])
            off += sz

    _per_tile(t, _zero_plan)
    plsc.subcore_barrier()

    def chunk(j, _):
        row0 = base_rows + j * KR
        ebase = t * EPT + j * KC
        pltpu.sync_copy(dst3.at[g, pl.ds(row0, KR)], dst_v)
        pltpu.sync_copy(e_all.at[0, g, pl.ds(ebase, KC)], e0_v)
        pltpu.sync_copy(e_all.at[1, g, pl.ds(ebase, KC)], e1_v)

        def fill(q, _):
            w0 = jnp.exp(e0_v[pl.ds(q * 16, 16)])
            w1 = jnp.exp(e1_v[pl.ds(q * 16, 16)])
            base = q * 16
            for m in range(16):
                i = base + m
                rows_v[i, pl.ds(0, 16)] = jnp.full((16,), w0[m], jnp.float32)
                rows_v[i, pl.ds(16, 16)] = jnp.full((16,), w1[m], jnp.float32)
            return 0

        lax.fori_loop(0, KC // 16, fill, 0)

        for r in range(KR):
            pltpu.sync_copy(rows_v.at[pl.ds(r * KB, KB)],
                            den_sh.at[dst_v.at[r]], add=True)
        return 0

    lax.fori_loop(0, NCHUNK, chunk, 0)

    plsc.subcore_barrier()

    def _out_plan(nb, sizes):
        off = 0
        for sz in sizes:
            pltpu.sync_copy(den_sh.at[pl.ds(nb + off, sz)],
                            dd_hbm.at[g, pl.ds(nb + off, sz)])
            off += sz

    _per_tile(t, _out_plan)


def _make_den_call():
    return functools.partial(
        pl.kernel,
        mesh=_mesh,
        compiler_params=pltpu.CompilerParams(use_tc_tiling_on_sc=False),
        out_type=jax.ShapeDtypeStruct((NG, N_PAD, C_DIM), jnp.float32),
        scratch_types=[
            pltpu.VMEM_SHARED((N_PAD, C_DIM), jnp.float32),
            pltpu.VMEM((KR, KB), jnp.int32),
            pltpu.VMEM((KC,), jnp.float32),
            pltpu.VMEM((KC,), jnp.float32),
            pltpu.VMEM((KC, C_DIM), jnp.float32),
        ],
    )(_den_kernel)


def _prop_kernel(h_hbm, src3, dst3, e_l, out_hbm,
                 out_sh, src_v, dst_v, e_v, rows_v, gsem):
    g = lax.axis_index("c")
    t = lax.axis_index("s")
    base_rows = t * ROWS_PT

    _zero_rows(rows_v, KC)

    def _init_plan(nb, sizes):
        off = 0
        for sz in sizes:
            pltpu.sync_copy(rows_v.at[pl.ds(0, sz)],
                            out_sh.at[pl.ds(nb + off, sz)])
            off += sz

    _per_tile(t, _init_plan)
    plsc.subcore_barrier()

    def chunk(j, _):
        row0 = base_rows + j * KR
        ebase = t * EPT + j * KC
        pltpu.sync_copy(src3.at[g, pl.ds(row0, KR)], src_v)
        pltpu.sync_copy(dst3.at[g, pl.ds(row0, KR)], dst_v)
        pltpu.sync_copy(e_l.at[g, pl.ds(ebase, KC)], e_v)

        ghandles = [
            pltpu.async_copy(h_hbm.at[src_v.at[r]],
                             rows_v.at[pl.ds(r * KB, KB)], gsem)
            for r in range(KR)
        ]
        for h in ghandles:
            h.wait()

        def scale(q, _):
            w16 = jnp.exp(e_v[pl.ds(q * 16, 16)])
            base = q * 16
            for m in range(16):
                i = base + m
                w = w16[m]
                rows_v[i, pl.ds(0, 16)] = rows_v[i, pl.ds(0, 16)] * w
                rows_v[i, pl.ds(16, 16)] = rows_v[i, pl.ds(16, 16)] * w
            return 0

        lax.fori_loop(0, KC // 16, scale, 0)

        for r in range(KR):
            pltpu.sync_copy(rows_v.at[pl.ds(r * KB, KB)],
                            out_sh.at[dst_v.at[r]], add=True)
        return 0

    lax.fori_loop(0, NCHUNK, chunk, 0)

    plsc.subcore_barrier()

    def _out_plan(nb, sizes):
        off = 0
        for sz in sizes:
            pltpu.sync_copy(out_sh.at[pl.ds(nb + off, sz)],
                            out_hbm.at[g, pl.ds(nb + off, sz)])
            off += sz

    _per_tile(t, _out_plan)


def _make_prop_call():
    return functools.partial(
        pl.kernel,
        mesh=_mesh,
        compiler_params=pltpu.CompilerParams(use_tc_tiling_on_sc=False),
        out_type=jax.ShapeDtypeStruct((NG, N_PAD, C_DIM), jnp.float32),
        scratch_types=[
            pltpu.VMEM_SHARED((N_PAD, C_DIM), jnp.float32),
            pltpu.VMEM((KR, KB), jnp.int32),
            pltpu.VMEM((KR, KB), jnp.int32),
            pltpu.VMEM((KC,), jnp.float32),
            pltpu.VMEM((KC, C_DIM), jnp.float32),
            pltpu.SemaphoreType.DMA,
        ],
    )(_prop_kernel)


BN = 1000


def _make_epi_body(col):
    def _epi_body(p_ref, dd_ref, m_ref, loh_ref, o_ref):
        den = dd_ref[0][:, col:col + 1]
        p = p_ref[0]
        q = jnp.where(den > 0.0, p / den, 0.0)
        m = m_ref[0]
        o_ref[0] = jnp.maximum(q, 0.0) * (1.0 - m) + loh_ref[0] * m
    return _epi_body


def _epilogue(p, dd, col, mask_f, loh):
    grid = (NG, N_NODES // BN)
    return pl.pallas_call(
        _make_epi_body(col),
        grid=grid,
        in_specs=[
            pl.BlockSpec((1, BN, C_DIM), lambda g, b: (g, b, 0)),
            pl.BlockSpec((1, BN, C_DIM), lambda g, b: (g, b, 0)),
            pl.BlockSpec((1, BN, 1), lambda g, b: (0, b, 0)),
            pl.BlockSpec((1, BN, C_DIM), lambda g, b: (0, b, 0)),
        ],
        out_specs=pl.BlockSpec((1, BN, C_DIM), lambda g, b: (g, b, 0)),
        out_shape=jax.ShapeDtypeStruct((NG, N_NODES, C_DIM), jnp.float32),
    )(p, dd, mask_f[None], loh[None])


def _final_body(h2_ref, att_ref, alpha_ref, feat_ref, w1_ref, b1_ref,
                w2_ref, b2_ref, o_ref):
    feat = feat_ref[...]
    z = jnp.dot(feat, w1_ref[...], preferred_element_type=jnp.float32)
    z = jnp.maximum(z + b1_ref[...], 0.0)
    mlp = jnp.dot(z, w2_ref[...], preferred_element_type=jnp.float32)
    mlp = mlp + b2_ref[...]

    att = att_ref[...]
    amax = jnp.max(att, axis=1, keepdims=True)
    ae = jnp.exp(att - amax)
    aw = ae / jnp.sum(ae, axis=1, keepdims=True)

    logits = (h2_ref[0] * aw[:, 0:1] + h2_ref[1] * aw[:, 1:2])
    al = alpha_ref[...]
    sa = jax.nn.sigmoid(al)
    sb = jax.nn.sigmoid(-al)
    o_ref[...] = sa * logits + sb * mlp


def _final(h2, att, alpha, feat, w1, b1, w2, b2):
    grid = (N_NODES // BN,)
    return pl.pallas_call(
        _final_body,
        grid=grid,
        in_specs=[
            pl.BlockSpec((NG, BN, C_DIM), lambda b: (0, b, 0)),
            pl.BlockSpec((BN, NG), lambda b: (b, 0)),
            pl.BlockSpec((BN, 1), lambda b: (b, 0)),
            pl.BlockSpec((BN, D_DIM), lambda b: (b, 0)),
            pl.BlockSpec((D_DIM, H_DIM), lambda b: (0, 0)),
            pl.BlockSpec((1, H_DIM), lambda b: (0, 0)),
            pl.BlockSpec((H_DIM, C_DIM), lambda b: (0, 0)),
            pl.BlockSpec((1, C_DIM), lambda b: (0, 0)),
        ],
        out_specs=pl.BlockSpec((BN, C_DIM), lambda b: (b, 0)),
        out_shape=jax.ShapeDtypeStruct((N_NODES, C_DIM), jnp.float32),
    )(h2, att, alpha, feat, w1, b1, w2, b2)


def kernel(features, src, dst, e_params, alpha, attention, labels_init,
           labels_one_hot, byte_idx_train, W1, b1, W2, b2):
    pad = E_PAD - E_EDGES
    src_p = jnp.concatenate(
        [src, jnp.zeros((NG, pad), jnp.int32)], axis=1)
    dst_p = jnp.concatenate(
        [dst, jnp.zeros((NG, pad), jnp.int32)], axis=1)
    e_p = jnp.concatenate(
        [e_params[:, :, :, 0],
         jnp.full((NL, NG, pad), -1e30, jnp.float32)], axis=2)

    dst3 = dst_p.reshape(NG, E_PAD // KB, KB)
    src3 = src_p.reshape(NG, E_PAD // KB, KB)
    src3o = src3 + (jnp.arange(NG, dtype=jnp.int32) * N_NODES)[:, None, None]

    dd = _make_den_call()(dst3, e_p)
    ddn = dd[:, :N_NODES]

    mask_f = byte_idx_train.astype(jnp.float32)

    h0 = labels_init
    p0 = _make_prop_call()(h0, src3, dst3, e_p[0])
    h1 = _epilogue(p0[:, :N_NODES], ddn, 0, mask_f, labels_one_hot)

    h1f = h1.reshape(NG * N_NODES, C_DIM)
    p1 = _make_prop_call()(h1f, src3o, dst3, e_p[1])
    h2 = _epilogue(p1[:, :N_NODES], ddn, 16, mask_f, labels_one_hot)

    return _final(h2, attention[:, :, 0], alpha, features[0],
                  W1, b1[None], W2, b2[None])


# speedup vs baseline: 26.0805x; 1.1074x over previous
import functools

import jax
import jax.numpy as jnp
from jax import lax
from jax.experimental import pallas as pl
from jax.experimental.pallas import tpu as pltpu
from jax.experimental.pallas import tpu_sc as plsc

N_NODES = 50000
N_PAD = 50048
E_EDGES = 800000
C_DIM = 32
D_DIM = 128
H_DIM = 64
NG = 2
NL = 2

E_PAD = 819200
KB = 128
KC = 512
KR = KC // KB
EPT = E_PAD // 16
NCHUNK = EPT // KC
ROWS_PT = E_PAD // KB // 16

NB_A = 3200
NB_B = 3072
SZS_A = (512, 512, 512, 512, 512, 512, 128)
SZS_B = (512, 512, 512, 512, 512, 512)

PKC = 256
PKR = PKC // KB
PNCHUNK = EPT // PKC
PSZS_A = (256,) * 12 + (128,)
PSZS_B = (256,) * 12

_mesh = plsc.VectorSubcoreMesh(core_axis_name="c", subcore_axis_name="s")


def _zero_rows(buf, nrows):
    z = jnp.zeros((16,), jnp.float32)

    def body(i, _):
        buf[i, pl.ds(0, 16)] = z
        buf[i, pl.ds(16, 16)] = z
        return 0

    lax.fori_loop(0, nrows, body, 0)


def _per_tile(t, plan, a_sizes=SZS_A, b_sizes=SZS_B):
    @pl.when(t < 7)
    def _():
        plan(t * NB_A, a_sizes)

    @pl.when(t >= 7)
    def _():
        plan(7 * NB_A + (t - 7) * NB_B, b_sizes)


def _den_kernel(dst3, e_all, dd_hbm, den_sh, dst_v, e0_v, e1_v, rows_v):
    g = lax.axis_index("c")
    t = lax.axis_index("s")
    base_rows = t * ROWS_PT

    _zero_rows(rows_v, KC)

    def _zero_plan(nb, sizes):
        off = 0
        for sz in sizes:
            pltpu.sync_copy(rows_v.at[pl.ds(0, sz)],
                            den_sh.at[pl.ds(nb + off, sz)])
            off += sz

    _per_tile(t, _zero_plan)
    plsc.subcore_barrier()

    def chunk(j, _):
        row0 = base_rows + j * KR
        ebase = t * EPT + j * KC
        pltpu.sync_copy(dst3.at[g, pl.ds(row0, KR)], dst_v)
        pltpu.sync_copy(e_all.at[0, g, pl.ds(ebase, KC)], e0_v)
        pltpu.sync_copy(e_all.at[1, g, pl.ds(ebase, KC)], e1_v)

        def fill(q, _):
            w0 = jnp.exp(e0_v[pl.ds(q * 16, 16)])
            w1 = jnp.exp(e1_v[pl.ds(q * 16, 16)])
            base = q * 16
            for m in range(16):
                i = base + m
                rows_v[i, pl.ds(0, 16)] = jnp.full((16,), w0[m], jnp.float32)
                rows_v[i, pl.ds(16, 16)] = jnp.full((16,), w1[m], jnp.float32)
            return 0

        lax.fori_loop(0, KC // 16, fill, 0)

        for r in range(KR):
            pltpu.sync_copy(rows_v.at[pl.ds(r * KB, KB)],
                            den_sh.at[dst_v.at[r]], add=True)
        return 0

    lax.fori_loop(0, NCHUNK, chunk, 0)

    plsc.subcore_barrier()

    def _out_plan(nb, sizes):
        off = 0
        for sz in sizes:
            pltpu.sync_copy(den_sh.at[pl.ds(nb + off, sz)],
                            dd_hbm.at[g, pl.ds(nb + off, sz)])
            off += sz

    _per_tile(t, _out_plan)


def _make_den_call():
    return functools.partial(
        pl.kernel,
        mesh=_mesh,
        compiler_params=pltpu.CompilerParams(use_tc_tiling_on_sc=False),
        out_type=jax.ShapeDtypeStruct((NG, N_PAD, C_DIM), jnp.float32),
        scratch_types=[
            pltpu.VMEM_SHARED((N_PAD, C_DIM), jnp.float32),
            pltpu.VMEM((KR, KB), jnp.int32),
            pltpu.VMEM((KC,), jnp.float32),
            pltpu.VMEM((KC,), jnp.float32),
            pltpu.VMEM((KC, C_DIM), jnp.float32),
        ],
    )(_den_kernel)


def _prop_kernel(h_hbm, pk3, out_hbm,
                 out_sh, pk_v, rows_v, gsem0, gsem1, ssem0, ssem1):
    g = lax.axis_index("c")
    t = lax.axis_index("s")
    base_rows = t * ROWS_PT
    gsems = (gsem0, gsem1)
    ssems = (ssem0, ssem1)

    _zero_rows(rows_v.at[0], PKC)

    def _init_plan(nb, sizes):
        off = 0
        for sz in sizes:
            pltpu.sync_copy(rows_v.at[0, pl.ds(0, sz)],
                            out_sh.at[pl.ds(nb + off, sz)])
            off += sz

    _per_tile(t, _init_plan, a_sizes=PSZS_A, b_sizes=PSZS_B)
    plsc.subcore_barrier()

    def _load_pack(slot, j):
        pltpu.sync_copy(pk3.at[g, pl.ds(base_rows + j * PKR, PKR)],
                        pk_v.at[slot])

    def _issue_gathers(slot):
        for r in range(PKR):
            pltpu.async_copy(h_hbm.at[pk_v.at[slot, r, 0]],
                             rows_v.at[slot, pl.ds(r * KB, KB)], gsems[slot])

    def _drain_gathers(slot):
        for r in range(PKR):
            pltpu.make_async_copy(h_hbm.at[pk_v.at[slot, r, 0]],
                                  rows_v.at[slot, pl.ds(r * KB, KB)],
                                  gsems[slot]).wait()

    def _issue_scatters(slot):
        for r in range(PKR):
            pltpu.async_copy(rows_v.at[slot, pl.ds(r * KB, KB)],
                             out_sh.at[pk_v.at[slot, r, 1]], ssems[slot],
                             add=True)

    def _drain_scatters(slot):
        for r in range(PKR):
            pltpu.make_async_copy(rows_v.at[slot, pl.ds(r * KB, KB)],
                                  out_sh.at[pk_v.at[slot, r, 1]],
                                  ssems[slot]).wait()

    for slot in (0, 1):
        _load_pack(slot, slot)
        _issue_gathers(slot)

    def _body(jj, slot):
        j = 2 * jj + slot
        _drain_gathers(slot)

        for r in range(PKR):
            def grp(m, _):
                ei = pk_v[slot, r, 2, pl.ds(m * 16, 16)]
                w16 = jnp.exp(plsc.bitcast(ei, jnp.float32))
                for mm in range(16):
                    i = r * KB + m * 16 + mm
                    w = w16[mm]
                    rows_v[slot, i, pl.ds(0, 16)] = (
                        rows_v[slot, i, pl.ds(0, 16)] * w)
                    rows_v[slot, i, pl.ds(16, 16)] = (
                        rows_v[slot, i, pl.ds(16, 16)] * w)
                return 0

            lax.fori_loop(0, KB // 16, grp, 0)

        _issue_scatters(slot)

        @pl.when(j + 2 < PNCHUNK)
        def _():
            _drain_scatters(slot)
            _load_pack(slot, j + 2)
            _issue_gathers(slot)

    def loop(jj, _):
        _body(jj, 0)
        _body(jj, 1)
        return 0

    lax.fori_loop(0, PNCHUNK // 2, loop, 0)

    for slot in (0, 1):
        _drain_scatters(slot)

    plsc.subcore_barrier()

    def _out_plan(nb, sizes):
        off = 0
        for sz in sizes:
            pltpu.sync_copy(out_sh.at[pl.ds(nb + off, sz)],
                            out_hbm.at[g, pl.ds(nb + off, sz)])
            off += sz

    _per_tile(t, _out_plan)


def _make_prop_call():
    return functools.partial(
        pl.kernel,
        mesh=_mesh,
        compiler_params=pltpu.CompilerParams(
            use_tc_tiling_on_sc=False, needs_layout_passes=False),
        out_type=jax.ShapeDtypeStruct((NG, N_PAD, C_DIM), jnp.float32),
        scratch_types=[
            pltpu.VMEM_SHARED((N_PAD, C_DIM), jnp.float32),
            pltpu.VMEM((2, PKR, 3, KB), jnp.int32),
            pltpu.VMEM((2, PKC, C_DIM), jnp.float32),
            pltpu.SemaphoreType.DMA,
            pltpu.SemaphoreType.DMA,
            pltpu.SemaphoreType.DMA,
            pltpu.SemaphoreType.DMA,
        ],
    )(_prop_kernel)


BN = 1000


def _make_epi_body(col):
    def _epi_body(p_ref, dd_ref, m_ref, loh_ref, o_ref):
        den = dd_ref[0][:, col:col + 1]
        p = p_ref[0]
        q = jnp.where(den > 0.0, p / den, 0.0)
        m = m_ref[0]
        o_ref[0] = jnp.maximum(q, 0.0) * (1.0 - m) + loh_ref[0] * m
    return _epi_body


def _epilogue(p, dd, col, mask_f, loh):
    grid = (NG, N_NODES // BN)
    return pl.pallas_call(
        _make_epi_body(col),
        grid=grid,
        in_specs=[
            pl.BlockSpec((1, BN, C_DIM), lambda g, b: (g, b, 0)),
            pl.BlockSpec((1, BN, C_DIM), lambda g, b: (g, b, 0)),
            pl.BlockSpec((1, BN, 1), lambda g, b: (0, b, 0)),
            pl.BlockSpec((1, BN, C_DIM), lambda g, b: (0, b, 0)),
        ],
        out_specs=pl.BlockSpec((1, BN, C_DIM), lambda g, b: (g, b, 0)),
        out_shape=jax.ShapeDtypeStruct((NG, N_NODES, C_DIM), jnp.float32),
    )(p, dd, mask_f[None], loh[None])


def _final_body(h2_ref, att_ref, alpha_ref, feat_ref, w1_ref, b1_ref,
                w2_ref, b2_ref, o_ref):
    feat = feat_ref[...]
    z = jnp.dot(feat, w1_ref[...], preferred_element_type=jnp.float32)
    z = jnp.maximum(z + b1_ref[...], 0.0)
    mlp = jnp.dot(z, w2_ref[...], preferred_element_type=jnp.float32)
    mlp = mlp + b2_ref[...]

    att = att_ref[...]
    amax = jnp.max(att, axis=1, keepdims=True)
    ae = jnp.exp(att - amax)
    aw = ae / jnp.sum(ae, axis=1, keepdims=True)

    logits = (h2_ref[0] * aw[:, 0:1] + h2_ref[1] * aw[:, 1:2])
    al = alpha_ref[...]
    sa = jax.nn.sigmoid(al)
    sb = jax.nn.sigmoid(-al)
    o_ref[...] = sa * logits + sb * mlp


def _final(h2, att, alpha, feat, w1, b1, w2, b2):
    grid = (N_NODES // BN,)
    return pl.pallas_call(
        _final_body,
        grid=grid,
        in_specs=[
            pl.BlockSpec((NG, BN, C_DIM), lambda b: (0, b, 0)),
            pl.BlockSpec((BN, NG), lambda b: (b, 0)),
            pl.BlockSpec((BN, 1), lambda b: (b, 0)),
            pl.BlockSpec((BN, D_DIM), lambda b: (b, 0)),
            pl.BlockSpec((D_DIM, H_DIM), lambda b: (0, 0)),
            pl.BlockSpec((1, H_DIM), lambda b: (0, 0)),
            pl.BlockSpec((H_DIM, C_DIM), lambda b: (0, 0)),
            pl.BlockSpec((1, C_DIM), lambda b: (0, 0)),
        ],
        out_specs=pl.BlockSpec((BN, C_DIM), lambda b: (b, 0)),
        out_shape=jax.ShapeDtypeStruct((N_NODES, C_DIM), jnp.float32),
    )(h2, att, alpha, feat, w1, b1, w2, b2)


def kernel(features, src, dst, e_params, alpha, attention, labels_init,
           labels_one_hot, byte_idx_train, W1, b1, W2, b2):
    pad = E_PAD - E_EDGES
    src_p = jnp.concatenate(
        [src, jnp.zeros((NG, pad), jnp.int32)], axis=1)
    dst_p = jnp.concatenate(
        [dst, jnp.zeros((NG, pad), jnp.int32)], axis=1)
    e_p = jnp.concatenate(
        [e_params[:, :, :, 0],
         jnp.full((NL, NG, pad), -1e30, jnp.float32)], axis=2)

    dst3 = dst_p.reshape(NG, E_PAD // KB, KB)
    src3 = src_p.reshape(NG, E_PAD // KB, KB)
    src3o = src3 + (jnp.arange(NG, dtype=jnp.int32) * N_NODES)[:, None, None]
    e3i = lax.bitcast_convert_type(e_p, jnp.int32).reshape(
        NL, NG, E_PAD // KB, KB)
    pk_l0 = jnp.stack([src3, dst3, e3i[0]], axis=2)
    pk_l1 = jnp.stack([src3o, dst3, e3i[1]], axis=2)

    dd = _make_den_call()(dst3, e_p)
    ddn = dd[:, :N_NODES]

    mask_f = byte_idx_train.astype(jnp.float32)

    h0 = labels_init
    p0 = _make_prop_call()(h0, pk_l0)
    h1 = _epilogue(p0[:, :N_NODES], ddn, 0, mask_f, labels_one_hot)

    h1f = h1.reshape(NG * N_NODES, C_DIM)
    p1 = _make_prop_call()(h1f, pk_l1)
    h2 = _epilogue(p1[:, :N_NODES], ddn, 16, mask_f, labels_one_hot)

    return _final(h2, attention[:, :, 0], alpha, features[0],
                  W1, b1[None], W2, b2[None])


# speedup vs baseline: 29.3159x; 1.1241x over previous
import functools

import jax
import jax.numpy as jnp
from jax import lax
from jax.experimental import pallas as pl
from jax.experimental.pallas import tpu as pltpu
from jax.experimental.pallas import tpu_sc as plsc

N_NODES = 50000
N_PAD = 50048
E_EDGES = 800000
C_DIM = 32
D_DIM = 128
H_DIM = 64
NG = 2
NL = 2

E_PAD = 819200
KB = 128
KC = 512
KR = KC // KB
EPT = E_PAD // 16
NCHUNK = EPT // KC
ROWS_PT = E_PAD // KB // 16

NB_A = 3200
NB_B = 3072
SZS_A = (512, 512, 512, 512, 512, 512, 128)
SZS_B = (512, 512, 512, 512, 512, 512)

PKC = 256
PKR = PKC // KB
PNCHUNK = EPT // PKC
PSZS_A = (256,) * 12 + (128,)
PSZS_B = (256,) * 12

_mesh = plsc.VectorSubcoreMesh(core_axis_name="c", subcore_axis_name="s")


def _zero_rows(buf, nrows):
    z = jnp.zeros((16,), jnp.float32)

    def body(i, _):
        buf[i, pl.ds(0, 16)] = z
        buf[i, pl.ds(16, 16)] = z
        return 0

    lax.fori_loop(0, nrows, body, 0)


def _per_tile(t, plan, a_sizes=SZS_A, b_sizes=SZS_B):
    @pl.when(t < 7)
    def _():
        plan(t * NB_A, a_sizes)

    @pl.when(t >= 7)
    def _():
        plan(7 * NB_A + (t - 7) * NB_B, b_sizes)


def _den_kernel(pkd, dd_hbm, den_sh, pk_v, drows, ssem0, ssem1):
    g = lax.axis_index("c")
    t = lax.axis_index("s")
    base_rows = t * ROWS_PT
    ssems = (ssem0, ssem1)
    lane_lo = lax.broadcasted_iota(jnp.int32, (16,), 0) < 8

    def zrow(i, _):
        drows[0, i, pl.ds(0, 16)] = jnp.zeros((16,), jnp.float32)
        return 0

    lax.fori_loop(0, PKC, zrow, 0)

    def _init_plan(nb, sizes):
        off = 0
        for sz in sizes:
            pltpu.sync_copy(drows.at[0, pl.ds(0, sz)],
                            den_sh.at[pl.ds(nb + off, sz)])
            off += sz

    _per_tile(t, _init_plan, a_sizes=PSZS_A, b_sizes=PSZS_B)
    plsc.subcore_barrier()

    def _load_pack(slot, j):
        pltpu.sync_copy(pkd.at[g, pl.ds(base_rows + j * PKR, PKR)],
                        pk_v.at[slot])

    def _issue_scatters(slot):
        for r in range(PKR):
            pltpu.async_copy(drows.at[slot, pl.ds(r * KB, KB)],
                             den_sh.at[pk_v.at[slot, r, 0]], ssems[slot],
                             add=True)

    def _drain_scatters(slot):
        for r in range(PKR):
            pltpu.make_async_copy(drows.at[slot, pl.ds(r * KB, KB)],
                                  den_sh.at[pk_v.at[slot, r, 0]],
                                  ssems[slot]).wait()

    for slot in (0, 1):
        _load_pack(slot, slot)

    def _body(jj, slot):
        j = 2 * jj + slot

        for r in range(PKR):
            def grp(m, _):
                w0 = jnp.exp(plsc.bitcast(
                    pk_v[slot, r, 1, pl.ds(m * 16, 16)], jnp.float32))
                w1 = jnp.exp(plsc.bitcast(
                    pk_v[slot, r, 2, pl.ds(m * 16, 16)], jnp.float32))
                for mm in range(16):
                    i = r * KB + m * 16 + mm
                    v = jnp.where(lane_lo,
                                  jnp.full((16,), w0[mm], jnp.float32),
                                  jnp.full((16,), w1[mm], jnp.float32))
                    drows[slot, i, pl.ds(0, 16)] = v
                return 0

            lax.fori_loop(0, KB // 16, grp, 0)

        _issue_scatters(slot)

        @pl.when(j + 2 < PNCHUNK)
        def _():
            _drain_scatters(slot)
            _load_pack(slot, j + 2)

    def loop(jj, _):
        _body(jj, 0)
        _body(jj, 1)
        return 0

    lax.fori_loop(0, PNCHUNK // 2, loop, 0)

    for slot in (0, 1):
        _drain_scatters(slot)

    plsc.subcore_barrier()

    def _out_plan(nb, sizes):
        off = 0
        for sz in sizes:
            pltpu.sync_copy(den_sh.at[pl.ds(nb + off, sz)],
                            dd_hbm.at[g, pl.ds(nb + off, sz)])
            off += sz

    _per_tile(t, _out_plan)


def _make_den_call():
    return functools.partial(
        pl.kernel,
        mesh=_mesh,
        compiler_params=pltpu.CompilerParams(
            use_tc_tiling_on_sc=False, needs_layout_passes=False),
        out_type=jax.ShapeDtypeStruct((NG, N_PAD, 16), jnp.float32),
        scratch_types=[
            pltpu.VMEM_SHARED((N_PAD, 16), jnp.float32),
            pltpu.VMEM((2, PKR, 3, KB), jnp.int32),
            pltpu.VMEM((2, PKC, 16), jnp.float32),
            pltpu.SemaphoreType.DMA,
            pltpu.SemaphoreType.DMA,
        ],
    )(_den_kernel)


def _prop_kernel(h_hbm, pk3, out_hbm,
                 out_sh, pk_v, rows_v, gsem0, gsem1, ssem0, ssem1):
    g = lax.axis_index("c")
    t = lax.axis_index("s")
    base_rows = t * ROWS_PT
    gsems = (gsem0, gsem1)
    ssems = (ssem0, ssem1)

    _zero_rows(rows_v.at[0], PKC)

    def _init_plan(nb, sizes):
        off = 0
        for sz in sizes:
            pltpu.sync_copy(rows_v.at[0, pl.ds(0, sz)],
                            out_sh.at[pl.ds(nb + off, sz)])
            off += sz

    _per_tile(t, _init_plan, a_sizes=PSZS_A, b_sizes=PSZS_B)
    plsc.subcore_barrier()

    def _load_pack(slot, j):
        pltpu.sync_copy(pk3.at[g, pl.ds(base_rows + j * PKR, PKR)],
                        pk_v.at[slot])

    def _issue_gathers(slot):
        for r in range(PKR):
            pltpu.async_copy(h_hbm.at[pk_v.at[slot, r, 0]],
                             rows_v.at[slot, pl.ds(r * KB, KB)], gsems[slot])

    def _drain_gathers(slot):
        for r in range(PKR):
            pltpu.make_async_copy(h_hbm.at[pk_v.at[slot, r, 0]],
                                  rows_v.at[slot, pl.ds(r * KB, KB)],
                                  gsems[slot]).wait()

    def _issue_scatters(slot):
        for r in range(PKR):
            pltpu.async_copy(rows_v.at[slot, pl.ds(r * KB, KB)],
                             out_sh.at[pk_v.at[slot, r, 1]], ssems[slot],
                             add=True)

    def _drain_scatters(slot):
        for r in range(PKR):
            pltpu.make_async_copy(rows_v.at[slot, pl.ds(r * KB, KB)],
                                  out_sh.at[pk_v.at[slot, r, 1]],
                                  ssems[slot]).wait()

    for slot in (0, 1):
        _load_pack(slot, slot)
        _issue_gathers(slot)

    def _body(jj, slot):
        j = 2 * jj + slot
        _drain_gathers(slot)

        for r in range(PKR):
            def grp(m, _):
                ei = pk_v[slot, r, 2, pl.ds(m * 16, 16)]
                w16 = jnp.exp(plsc.bitcast(ei, jnp.float32))
                for mm in range(16):
                    i = r * KB + m * 16 + mm
                    w = w16[mm]
                    rows_v[slot, i, pl.ds(0, 16)] = (
                        rows_v[slot, i, pl.ds(0, 16)] * w)
                    rows_v[slot, i, pl.ds(16, 16)] = (
                        rows_v[slot, i, pl.ds(16, 16)] * w)
                return 0

            lax.fori_loop(0, KB // 16, grp, 0)

        _issue_scatters(slot)

        @pl.when(j + 2 < PNCHUNK)
        def _():
            _drain_scatters(slot)
            _load_pack(slot, j + 2)
            _issue_gathers(slot)

    def loop(jj, _):
        _body(jj, 0)
        _body(jj, 1)
        return 0

    lax.fori_loop(0, PNCHUNK // 2, loop, 0)

    for slot in (0, 1):
        _drain_scatters(slot)

    plsc.subcore_barrier()

    def _out_plan(nb, sizes):
        off = 0
        for sz in sizes:
            pltpu.sync_copy(out_sh.at[pl.ds(nb + off, sz)],
                            out_hbm.at[g, pl.ds(nb + off, sz)])
            off += sz

    _per_tile(t, _out_plan)


def _make_prop_call():
    return functools.partial(
        pl.kernel,
        mesh=_mesh,
        compiler_params=pltpu.CompilerParams(
            use_tc_tiling_on_sc=False, needs_layout_passes=False),
        out_type=jax.ShapeDtypeStruct((NG, N_PAD, C_DIM), jnp.float32),
        scratch_types=[
            pltpu.VMEM_SHARED((N_PAD, C_DIM), jnp.float32),
            pltpu.VMEM((2, PKR, 3, KB), jnp.int32),
            pltpu.VMEM((2, PKC, C_DIM), jnp.float32),
            pltpu.SemaphoreType.DMA,
            pltpu.SemaphoreType.DMA,
            pltpu.SemaphoreType.DMA,
            pltpu.SemaphoreType.DMA,
        ],
    )(_prop_kernel)


BN = 1000


def _make_epi_body(col):
    def _epi_body(p_ref, dd_ref, m_ref, loh_ref, o_ref):
        den = dd_ref[0][:, col:col + 1]
        p = p_ref[0]
        q = jnp.where(den > 0.0, p / den, 0.0)
        m = m_ref[0]
        o_ref[0] = jnp.maximum(q, 0.0) * (1.0 - m) + loh_ref[0] * m
    return _epi_body


def _epilogue(p, dd, col, mask_f, loh):
    grid = (NG, N_NODES // BN)
    return pl.pallas_call(
        _make_epi_body(col),
        grid=grid,
        in_specs=[
            pl.BlockSpec((1, BN, C_DIM), lambda g, b: (g, b, 0)),
            pl.BlockSpec((1, BN, 16), lambda g, b: (g, b, 0)),
            pl.BlockSpec((1, BN, 1), lambda g, b: (0, b, 0)),
            pl.BlockSpec((1, BN, C_DIM), lambda g, b: (0, b, 0)),
        ],
        out_specs=pl.BlockSpec((1, BN, C_DIM), lambda g, b: (g, b, 0)),
        out_shape=jax.ShapeDtypeStruct((NG, N_NODES, C_DIM), jnp.float32),
    )(p, dd, mask_f[None], loh[None])


def _final_body(h2_ref, att_ref, alpha_ref, feat_ref, w1_ref, b1_ref,
                w2_ref, b2_ref, o_ref):
    feat = feat_ref[...]
    z = jnp.dot(feat, w1_ref[...], preferred_element_type=jnp.float32)
    z = jnp.maximum(z + b1_ref[...], 0.0)
    mlp = jnp.dot(z, w2_ref[...], preferred_element_type=jnp.float32)
    mlp = mlp + b2_ref[...]

    att = att_ref[...]
    amax = jnp.max(att, axis=1, keepdims=True)
    ae = jnp.exp(att - amax)
    aw = ae / jnp.sum(ae, axis=1, keepdims=True)

    logits = (h2_ref[0] * aw[:, 0:1] + h2_ref[1] * aw[:, 1:2])
    al = alpha_ref[...]
    sa = jax.nn.sigmoid(al)
    sb = jax.nn.sigmoid(-al)
    o_ref[...] = sa * logits + sb * mlp


def _final(h2, att, alpha, feat, w1, b1, w2, b2):
    grid = (N_NODES // BN,)
    return pl.pallas_call(
        _final_body,
        grid=grid,
        in_specs=[
            pl.BlockSpec((NG, BN, C_DIM), lambda b: (0, b, 0)),
            pl.BlockSpec((BN, NG), lambda b: (b, 0)),
            pl.BlockSpec((BN, 1), lambda b: (b, 0)),
            pl.BlockSpec((BN, D_DIM), lambda b: (b, 0)),
            pl.BlockSpec((D_DIM, H_DIM), lambda b: (0, 0)),
            pl.BlockSpec((1, H_DIM), lambda b: (0, 0)),
            pl.BlockSpec((H_DIM, C_DIM), lambda b: (0, 0)),
            pl.BlockSpec((1, C_DIM), lambda b: (0, 0)),
        ],
        out_specs=pl.BlockSpec((BN, C_DIM), lambda b: (b, 0)),
        out_shape=jax.ShapeDtypeStruct((N_NODES, C_DIM), jnp.float32),
    )(h2, att, alpha, feat, w1, b1, w2, b2)


def kernel(features, src, dst, e_params, alpha, attention, labels_init,
           labels_one_hot, byte_idx_train, W1, b1, W2, b2):
    pad = E_PAD - E_EDGES
    src_p = jnp.concatenate(
        [src, jnp.zeros((NG, pad), jnp.int32)], axis=1)
    dst_p = jnp.concatenate(
        [dst, jnp.zeros((NG, pad), jnp.int32)], axis=1)
    e_p = jnp.concatenate(
        [e_params[:, :, :, 0],
         jnp.full((NL, NG, pad), -1e30, jnp.float32)], axis=2)

    dst3 = dst_p.reshape(NG, E_PAD // KB, KB)
    src3 = src_p.reshape(NG, E_PAD // KB, KB)
    src3o = src3 + (jnp.arange(NG, dtype=jnp.int32) * N_NODES)[:, None, None]
    e3i = lax.bitcast_convert_type(e_p, jnp.int32).reshape(
        NL, NG, E_PAD // KB, KB)
    pk_l0 = jnp.stack([src3o, dst3, e3i[0]], axis=2)
    pk_l1 = jnp.stack([src3o, dst3, e3i[1]], axis=2)
    pk_den = jnp.stack([dst3, e3i[0], e3i[1]], axis=2)

    dd = _make_den_call()(pk_den)
    ddn = dd[:, :N_NODES]

    mask_f = byte_idx_train.astype(jnp.float32)

    h0f = jnp.concatenate([labels_init, labels_init], axis=0)
    p0 = _make_prop_call()(h0f, pk_l0)
    h1 = _epilogue(p0[:, :N_NODES], ddn, 0, mask_f, labels_one_hot)

    h1f = h1.reshape(NG * N_NODES, C_DIM)
    p1 = _make_prop_call()(h1f, pk_l1)
    h2 = _epilogue(p1[:, :N_NODES], ddn, 8, mask_f, labels_one_hot)

    return _final(h2, attention[:, :, 0], alpha, features[0],
                  W1, b1[None], W2, b2[None])
